# baseline (speedup 1.0000x reference)
import jax
import jax.numpy as jnp
from jax import lax
from jax.experimental import pallas as pl
from jax.experimental.pallas import tpu as pltpu
from jax.experimental.pallas import tpu_sc as plsc

NUM_CLASSES = 5
OUTP = 8
LAT = 16
BATCH = 16384
NC, NS, L = 2, 16, 16
NW = NC * NS
BPW = BATCH // NW
NBLK = BPW // L


def _body(user_hbm, movie_hbm, utmf_hbm, mtmf_hbm, ut_hbm, mt_hbm,
          fcw_hbm, fcb_hbm, out_hbm,
          idx_u, idx_m, umf_v, mmf_v, u_v, m_v, w_v, b_v, acc_v, sem):
    wid = lax.axis_index("s") * NC + lax.axis_index("c")
    base = wid * BPW

    pltpu.sync_copy(user_hbm.at[pl.ds(base, BPW)], idx_u)
    pltpu.sync_copy(movie_hbm.at[pl.ds(base, BPW)], idx_m)
    pltpu.sync_copy(fcw_hbm, w_v)
    pltpu.sync_copy(fcb_hbm, b_v)

    copies = []
    for k in range(LAT):
        copies.append(pltpu.async_copy(utmf_hbm.at[k].at[idx_u], umf_v.at[k], sem))
        copies.append(pltpu.async_copy(mtmf_hbm.at[k].at[idx_m], mmf_v.at[k], sem))
        copies.append(pltpu.async_copy(ut_hbm.at[k].at[idx_u], u_v.at[k], sem))
        copies.append(pltpu.async_copy(mt_hbm.at[k].at[idx_m], m_v.at[k], sem))
    for c in copies:
        c.wait()

    bias = b_v[...]

    def init_body(blk, carry):
        s = pl.ds(blk * L, L)
        for c in range(NUM_CLASSES):
            acc_v[c, s] = jnp.full((L,), bias[c], jnp.float32)
        return carry

    lax.fori_loop(0, NBLK, init_body, 0)

    w_mf = [w_v[c, 0:LAT] for c in range(NUM_CLASSES)]
    w_u = [w_v[c, LAT:2 * LAT] for c in range(NUM_CLASSES)]
    w_m = [w_v[c, 2 * LAT:3 * LAT] for c in range(NUM_CLASSES)]

    for k in range(LAT):
        wmf_k = [w_mf[c][k] for c in range(NUM_CLASSES)]
        wu_k = [w_u[c][k] for c in range(NUM_CLASSES)]
        wm_k = [w_m[c][k] for c in range(NUM_CLASSES)]

        def k_body(blk, carry, k=k, wmf_k=wmf_k, wu_k=wu_k, wm_k=wm_k):
            s = pl.ds(blk * L, L)
            mf = umf_v[k, s] * mmf_v[k, s]
            u = u_v[k, s]
            m = m_v[k, s]
            for c in range(NUM_CLASSES):
                acc_v[c, s] = (acc_v[c, s] + mf * wmf_k[c] + u * wu_k[c]
                               + m * wm_k[c])
            return carry

        lax.fori_loop(0, NBLK, k_body, 0)

    pltpu.sync_copy(acc_v, out_hbm.at[:, pl.ds(base, BPW)])


def kernel(user, movie, user_table_mf, movie_table_mf, user_table,
           movie_table, fc_w, fc_b):
    fcb_pad = jnp.pad(fc_b, (0, L - NUM_CLASSES))
    run = pl.kernel(
        _body,
        out_type=jax.ShapeDtypeStruct((OUTP, BATCH), jnp.float32),
        mesh=plsc.VectorSubcoreMesh(core_axis_name="c", subcore_axis_name="s"),
        compiler_params=pltpu.CompilerParams(needs_layout_passes=False,
                                             use_tc_tiling_on_sc=False),
        scratch_types=[
            pltpu.VMEM((BPW,), jnp.int32),
            pltpu.VMEM((BPW,), jnp.int32),
            pltpu.VMEM((LAT, BPW), jnp.float32),
            pltpu.VMEM((LAT, BPW), jnp.float32),
            pltpu.VMEM((LAT, BPW), jnp.float32),
            pltpu.VMEM((LAT, BPW), jnp.float32),
            pltpu.VMEM((NUM_CLASSES, 3 * LAT), jnp.float32),
            pltpu.VMEM((L,), jnp.float32),
            pltpu.VMEM((OUTP, BPW), jnp.float32),
            pltpu.SemaphoreType.DMA,
        ],
    )
    out_t = run(user, movie, user_table_mf.T, movie_table_mf.T,
                user_table.T, movie_table.T, fc_w, fcb_pad)
    return out_t[:NUM_CLASSES, :].T


# speedup vs baseline: 20.0568x; 20.0568x over previous
import jax
import jax.numpy as jnp
from jax import lax
from jax.experimental import pallas as pl
from jax.experimental.pallas import tpu as pltpu
from jax.experimental.pallas import tpu_sc as plsc

NUM_CLASSES = 5
LAT = 16
BATCH = 16384
NC, NS, L = 2, 16, 16
NW = NC * NS
BPW = BATCH // NW
NBLK = BPW // L
SUB = 4
SLAB = 128


def _body(user_hbm, movie_hbm, utmf_hbm, mtmf_hbm, ut_hbm, mt_hbm,
          fcw_hbm, fcb_hbm, out_hbm, idx_u, idx_m, *rest):
    slabs = rest[:32]
    w_v, b_v, out_vt, sem0, sem1 = rest[32:]
    sems = [sem0, sem1]
    tabs = [utmf_hbm, mtmf_hbm, ut_hbm, mt_hbm]

    wid = lax.axis_index("s") * NC + lax.axis_index("c")
    base = wid * BPW

    pltpu.sync_copy(user_hbm.at[pl.ds(base, BPW)], idx_u)
    pltpu.sync_copy(movie_hbm.at[pl.ds(base, BPW)], idx_m)
    pltpu.sync_copy(fcw_hbm, w_v)
    pltpu.sync_copy(fcb_hbm, b_v)

    def slab_ref(slot, i, t):
        return slabs[slot * 16 + i * 4 + t]

    def fire(iv_u, iv_m, j, slot):
        for i in range(SUB):
            ru = iv_u[j * SUB + i]
            rm = iv_m[j * SUB + i]
            off_u = pl.multiple_of((ru >> 7) << 7, SLAB)
            off_m = pl.multiple_of((rm >> 7) << 7, SLAB)
            for t in range(4):
                off = off_u if t in (0, 2) else off_m
                pltpu.async_copy(tabs[t].at[:, pl.ds(off, SLAB)],
                                 slab_ref(slot, i, t), sems[slot])

    def drain(slot):
        for i in range(SUB):
            for t in range(4):
                pltpu.make_async_copy(tabs[t].at[:, pl.ds(0, SLAB)],
                                      slab_ref(slot, i, t), sems[slot]).wait()

    lane = lax.iota(jnp.int32, L)
    bias = b_v[...]

    iv_u0 = idx_u[pl.ds(0, L)]
    iv_m0 = idx_m[pl.ds(0, L)]
    fire(iv_u0, iv_m0, 0, 0)
    fire(iv_u0, iv_m0, 1, 1)

    def blk_body(blk, carry):
        sb = pl.ds(blk * L, L)
        iv_u = idx_u[sb]
        iv_m = idx_m[sb]
        nb = jnp.minimum(blk + 1, NBLK - 1)
        snb = pl.ds(nb * L, L)
        ivn_u = idx_u[snb]
        ivn_m = idx_m[snb]
        for j in range(4):
            slot = j & 1
            drain(slot)
            for i in range(SUB):
                ru = iv_u[j * SUB + i]
                rm = iv_m[j * SUB + i]
                cu = jnp.full((L,), ru & (SLAB - 1), jnp.int32)
                cm = jnp.full((L,), rm & (SLAB - 1), jnp.int32)
                umf = plsc.load_gather(slab_ref(slot, i, 0), [lane, cu])
                mmf = plsc.load_gather(slab_ref(slot, i, 1), [lane, cm])
                u = plsc.load_gather(slab_ref(slot, i, 2), [lane, cu])
                m = plsc.load_gather(slab_ref(slot, i, 3), [lane, cm])
                mf = umf * mmf
                acc = bias
                for c in range(NUM_CLASSES):
                    t = (mf * w_v[c, 0:LAT] + u * w_v[c, LAT:2 * LAT]
                         + m * w_v[c, 2 * LAT:3 * LAT])
                    s = jnp.sum(t)
                    acc = jnp.where(lane == c, acc + s, acc)
                bpos = jnp.full((L,), blk * L + j * SUB + i, jnp.int32)
                plsc.store_scatter(out_vt, [lane, bpos], acc)
            if j < 2:
                fire(iv_u, iv_m, j + 2, slot)
            else:

                @pl.when(blk < NBLK - 1)
                def _(j=j, slot=slot, ivn_u=ivn_u, ivn_m=ivn_m):
                    fire(ivn_u, ivn_m, j - 2, slot)

        return carry

    lax.fori_loop(0, NBLK, blk_body, 0)

    pltpu.sync_copy(out_vt, out_hbm.at[:, pl.ds(base, BPW)])


def kernel(user, movie, user_table_mf, movie_table_mf, user_table,
           movie_table, fc_w, fc_b):
    fcb_pad = jnp.pad(fc_b, (0, L - NUM_CLASSES))
    run = pl.kernel(
        _body,
        out_type=jax.ShapeDtypeStruct((L, BATCH), jnp.float32),
        mesh=plsc.VectorSubcoreMesh(core_axis_name="c", subcore_axis_name="s"),
        compiler_params=pltpu.CompilerParams(needs_layout_passes=False,
                                             use_tc_tiling_on_sc=True),
        scratch_types=[
            pltpu.VMEM((BPW,), jnp.int32),
            pltpu.VMEM((BPW,), jnp.int32),
        ] + [pltpu.VMEM((LAT, SLAB), jnp.float32)
             for _ in range(32)] + [
            pltpu.VMEM((NUM_CLASSES, 3 * LAT), jnp.float32),
            pltpu.VMEM((L,), jnp.float32),
            pltpu.VMEM((L, BPW), jnp.float32),
            pltpu.SemaphoreType.DMA,
            pltpu.SemaphoreType.DMA,
        ],
    )
    out_t = run(user, movie, user_table_mf.T, movie_table_mf.T,
                user_table.T, movie_table.T, fc_w, fcb_pad)
    return out_t[:NUM_CLASSES, :].T


# speedup vs baseline: 22.1589x; 1.1048x over previous
import jax
import jax.numpy as jnp
from jax import lax
from jax.experimental import pallas as pl
from jax.experimental.pallas import tpu as pltpu
from jax.experimental.pallas import tpu_sc as plsc

NUM_CLASSES = 5
LAT = 16
BATCH = 16384
NC, NS, L = 2, 16, 16
NW = NC * NS
BPW = BATCH // NW
NBLK = BPW // L
SUB = 2
SLOTS = 4
SPB = L // SUB
SLAB = 128


def _body(user_hbm, movie_hbm, utmf_hbm, mtmf_hbm, ut_hbm, mt_hbm,
          fcw_hbm, fcb_hbm, out_hbm, idx_u, idx_m, *rest):
    slabs = rest[:SLOTS * SUB * 4]
    w_v, b_v, out_vt, *sems = rest[SLOTS * SUB * 4:]
    tabs = [utmf_hbm, mtmf_hbm, ut_hbm, mt_hbm]

    wid = lax.axis_index("s") * NC + lax.axis_index("c")
    base = wid * BPW

    pltpu.sync_copy(user_hbm.at[pl.ds(base, BPW)], idx_u)
    pltpu.sync_copy(movie_hbm.at[pl.ds(base, BPW)], idx_m)
    pltpu.sync_copy(fcw_hbm, w_v)
    pltpu.sync_copy(fcb_hbm, b_v)

    def slab_ref(slot, i, t):
        return slabs[(slot * SUB + i) * 4 + t]

    def fire(iv_u, iv_m, j, slot):
        for i in range(SUB):
            ru = iv_u[j * SUB + i]
            rm = iv_m[j * SUB + i]
            off_u = pl.multiple_of((ru >> 7) << 7, SLAB)
            off_m = pl.multiple_of((rm >> 7) << 7, SLAB)
            for t in range(4):
                off = off_u if t in (0, 2) else off_m
                pltpu.async_copy(tabs[t].at[:, pl.ds(off, SLAB)],
                                 slab_ref(slot, i, t), sems[slot])

    def drain(slot):
        for i in range(SUB):
            for t in range(4):
                pltpu.make_async_copy(tabs[t].at[:, pl.ds(0, SLAB)],
                                      slab_ref(slot, i, t), sems[slot]).wait()

    lane = lax.iota(jnp.int32, L)
    bias = b_v[...]

    iv_u0 = idx_u[pl.ds(0, L)]
    iv_m0 = idx_m[pl.ds(0, L)]
    for j in range(SLOTS):
        fire(iv_u0, iv_m0, j, j)

    def blk_body(blk, carry):
        sb = pl.ds(blk * L, L)
        iv_u = idx_u[sb]
        iv_m = idx_m[sb]
        nb = jnp.minimum(blk + 1, NBLK - 1)
        snb = pl.ds(nb * L, L)
        ivn_u = idx_u[snb]
        ivn_m = idx_m[snb]
        for j in range(SPB):
            slot = j % SLOTS
            drain(slot)
            for i in range(SUB):
                ru = iv_u[j * SUB + i]
                rm = iv_m[j * SUB + i]
                cu = jnp.full((L,), ru & (SLAB - 1), jnp.int32)
                cm = jnp.full((L,), rm & (SLAB - 1), jnp.int32)
                umf = plsc.load_gather(slab_ref(slot, i, 0), [lane, cu])
                mmf = plsc.load_gather(slab_ref(slot, i, 1), [lane, cm])
                u = plsc.load_gather(slab_ref(slot, i, 2), [lane, cu])
                m = plsc.load_gather(slab_ref(slot, i, 3), [lane, cm])
                mf = umf * mmf
                acc = bias
                for c in range(NUM_CLASSES):
                    t = (mf * w_v[c, 0:LAT] + u * w_v[c, LAT:2 * LAT]
                         + m * w_v[c, 2 * LAT:3 * LAT])
                    s = jnp.sum(t)
                    acc = jnp.where(lane == c, acc + s, acc)
                bpos = jnp.full((L,), blk * L + j * SUB + i, jnp.int32)
                plsc.store_scatter(out_vt, [lane, bpos], acc)
            if j < SPB - SLOTS:
                fire(iv_u, iv_m, j + SLOTS, slot)
            else:

                @pl.when(blk < NBLK - 1)
                def _(j=j, slot=slot, ivn_u=ivn_u, ivn_m=ivn_m):
                    fire(ivn_u, ivn_m, j + SLOTS - SPB, slot)

        return carry

    lax.fori_loop(0, NBLK, blk_body, 0)

    pltpu.sync_copy(out_vt, out_hbm.at[:, pl.ds(base, BPW)])


def kernel(user, movie, user_table_mf, movie_table_mf, user_table,
           movie_table, fc_w, fc_b):
    fcb_pad = jnp.pad(fc_b, (0, L - NUM_CLASSES))
    run = pl.kernel(
        _body,
        out_type=jax.ShapeDtypeStruct((L, BATCH), jnp.float32),
        mesh=plsc.VectorSubcoreMesh(core_axis_name="c", subcore_axis_name="s"),
        compiler_params=pltpu.CompilerParams(needs_layout_passes=False,
                                             use_tc_tiling_on_sc=True),
        scratch_types=[
            pltpu.VMEM((BPW,), jnp.int32),
            pltpu.VMEM((BPW,), jnp.int32),
        ] + [pltpu.VMEM((LAT, SLAB), jnp.float32)
             for _ in range(SLOTS * SUB * 4)] + [
            pltpu.VMEM((NUM_CLASSES, 3 * LAT), jnp.float32),
            pltpu.VMEM((L,), jnp.float32),
            pltpu.VMEM((L, BPW), jnp.float32),
        ] + [pltpu.SemaphoreType.DMA for _ in range(SLOTS)],
    )
    out_t = run(user, movie, user_table_mf.T, movie_table_mf.T,
                user_table.T, movie_table.T, fc_w, fcb_pad)
    return out_t[:NUM_CLASSES, :].T


# speedup vs baseline: 24.1520x; 1.0899x over previous
import jax
import jax.numpy as jnp
from jax import lax
from jax.experimental import pallas as pl
from jax.experimental.pallas import tpu as pltpu
from jax.experimental.pallas import tpu_sc as plsc

NUM_CLASSES = 5
LAT = 16
BATCH = 16384
NC, NS, L = 2, 16, 16
NW = NC * NS
BPW = BATCH // NW
NBLK = BPW // L
SUB = 1
SLOTS = 8
SPB = L // SUB
SLAB = 128


def _body(user_hbm, movie_hbm, utmf_hbm, mtmf_hbm, ut_hbm, mt_hbm,
          fcw_hbm, fcb_hbm, out_hbm, idx_u, idx_m, *rest):
    slabs = rest[:SLOTS * SUB * 4]
    w_v, b_v, out_vt, *sems = rest[SLOTS * SUB * 4:]
    tabs = [utmf_hbm, mtmf_hbm, ut_hbm, mt_hbm]

    wid = lax.axis_index("s") * NC + lax.axis_index("c")
    base = wid * BPW

    pltpu.sync_copy(user_hbm.at[pl.ds(base, BPW)], idx_u)
    pltpu.sync_copy(movie_hbm.at[pl.ds(base, BPW)], idx_m)
    pltpu.sync_copy(fcw_hbm, w_v)
    pltpu.sync_copy(fcb_hbm, b_v)

    def slab_ref(slot, i, t):
        return slabs[(slot * SUB + i) * 4 + t]

    def fire(iv_u, iv_m, j, slot):
        for i in range(SUB):
            ru = iv_u[j * SUB + i]
            rm = iv_m[j * SUB + i]
            off_u = pl.multiple_of((ru >> 7) << 7, SLAB)
            off_m = pl.multiple_of((rm >> 7) << 7, SLAB)
            for t in range(4):
                off = off_u if t in (0, 2) else off_m
                pltpu.async_copy(tabs[t].at[:, pl.ds(off, SLAB)],
                                 slab_ref(slot, i, t), sems[slot])

    def drain(slot):
        for i in range(SUB):
            for t in range(4):
                pltpu.make_async_copy(tabs[t].at[:, pl.ds(0, SLAB)],
                                      slab_ref(slot, i, t), sems[slot]).wait()

    lane = lax.iota(jnp.int32, L)
    bias = b_v[...]

    iv_u0 = idx_u[pl.ds(0, L)]
    iv_m0 = idx_m[pl.ds(0, L)]
    for j in range(SLOTS):
        fire(iv_u0, iv_m0, j, j)

    def blk_body(blk, carry):
        sb = pl.ds(blk * L, L)
        iv_u = idx_u[sb]
        iv_m = idx_m[sb]
        nb = jnp.minimum(blk + 1, NBLK - 1)
        snb = pl.ds(nb * L, L)
        ivn_u = idx_u[snb]
        ivn_m = idx_m[snb]
        for j in range(SPB):
            slot = j % SLOTS
            drain(slot)
            for i in range(SUB):
                ru = iv_u[j * SUB + i]
                rm = iv_m[j * SUB + i]
                cu = jnp.full((L,), ru & (SLAB - 1), jnp.int32)
                cm = jnp.full((L,), rm & (SLAB - 1), jnp.int32)
                umf = plsc.load_gather(slab_ref(slot, i, 0), [lane, cu])
                mmf = plsc.load_gather(slab_ref(slot, i, 1), [lane, cm])
                u = plsc.load_gather(slab_ref(slot, i, 2), [lane, cu])
                m = plsc.load_gather(slab_ref(slot, i, 3), [lane, cm])
                mf = umf * mmf
                acc = bias
                for c in range(NUM_CLASSES):
                    t = (mf * w_v[c, 0:LAT] + u * w_v[c, LAT:2 * LAT]
                         + m * w_v[c, 2 * LAT:3 * LAT])
                    s = jnp.sum(t)
                    acc = jnp.where(lane == c, acc + s, acc)
                bpos = jnp.full((L,), blk * L + j * SUB + i, jnp.int32)
                plsc.store_scatter(out_vt, [lane, bpos], acc)
            if j < SPB - SLOTS:
                fire(iv_u, iv_m, j + SLOTS, slot)
            else:

                @pl.when(blk < NBLK - 1)
                def _(j=j, slot=slot, ivn_u=ivn_u, ivn_m=ivn_m):
                    fire(ivn_u, ivn_m, j + SLOTS - SPB, slot)

        return carry

    lax.fori_loop(0, NBLK, blk_body, 0)

    pltpu.sync_copy(out_vt, out_hbm.at[:, pl.ds(base, BPW)])


def kernel(user, movie, user_table_mf, movie_table_mf, user_table,
           movie_table, fc_w, fc_b):
    fcb_pad = jnp.pad(fc_b, (0, L - NUM_CLASSES))
    run = pl.kernel(
        _body,
        out_type=jax.ShapeDtypeStruct((L, BATCH), jnp.float32),
        mesh=plsc.VectorSubcoreMesh(core_axis_name="c", subcore_axis_name="s"),
        compiler_params=pltpu.CompilerParams(needs_layout_passes=False,
                                             use_tc_tiling_on_sc=True),
        scratch_types=[
            pltpu.VMEM((BPW,), jnp.int32),
            pltpu.VMEM((BPW,), jnp.int32),
        ] + [pltpu.VMEM((LAT, SLAB), jnp.float32)
             for _ in range(SLOTS * SUB * 4)] + [
            pltpu.VMEM((NUM_CLASSES, 3 * LAT), jnp.float32),
            pltpu.VMEM((L,), jnp.float32),
            pltpu.VMEM((L, BPW), jnp.float32),
        ] + [pltpu.SemaphoreType.DMA for _ in range(SLOTS)],
    )
    out_t = run(user, movie, user_table_mf.T, movie_table_mf.T,
                user_table.T, movie_table.T, fc_w, fcb_pad)
    return out_t[:NUM_CLASSES, :].T
